# baseline (speedup 1.0000x reference)
import functools
import math

import jax
import jax.numpy as jnp
import numpy as np
from jax import lax
from jax.experimental import pallas as pl
from jax.experimental.pallas import tpu as pltpu
from jax.experimental.pallas import tpu_sc as plsc

N = 10000
E = 160000
D = 256
NHEAD = 8
HDIM = D // NHEAD

NC = 2
NS = 16
HALF = D // 2
NP = 10112

_MESH = functools.partial(
    plsc.VectorSubcoreMesh, core_axis_name="c", subcore_axis_name="s",
    num_cores=NC, num_subcores=NS)


BN = 1000


def _qkv_body(x_ref, wq_ref, wk_ref, wv_ref, b_ref, q_ref, kv_ref):
    x = x_ref[...]
    b = b_ref[...]
    xq = jnp.dot(x, wq_ref[...], preferred_element_type=jnp.float32, precision=lax.Precision.HIGHEST) + b[0:1, :]
    xk = jnp.dot(x, wk_ref[...], preferred_element_type=jnp.float32, precision=lax.Precision.HIGHEST) + b[1:2, :]
    xv = jnp.dot(x, wv_ref[...], preferred_element_type=jnp.float32, precision=lax.Precision.HIGHEST) + b[2:3, :]
    q_ref[0] = xq[:, :HALF]
    q_ref[1] = xq[:, HALF:]
    kv_ref[0, :, :HALF] = xk[:, :HALF]
    kv_ref[0, :, HALF:] = xv[:, :HALF]
    kv_ref[1, :, :HALF] = xk[:, HALF:]
    kv_ref[1, :, HALF:] = xv[:, HALF:]


def _qkv_proj(feats, wqt, wkt, wvt, b3):
    grid = (N // BN,)
    spec_x = pl.BlockSpec((BN, D), lambda i: (i, 0))
    spec_w = pl.BlockSpec((D, D), lambda i: (0, 0))
    spec_b = pl.BlockSpec((3, D), lambda i: (0, 0))
    return pl.pallas_call(
        _qkv_body, grid=grid,
        in_specs=[spec_x, spec_w, spec_w, spec_w, spec_b],
        out_specs=[pl.BlockSpec((2, BN, HALF), lambda i: (0, i, 0)),
                   pl.BlockSpec((2, BN, D), lambda i: (0, i, 0))],
        out_shape=[jax.ShapeDtypeStruct((2, N, HALF), jnp.float32),
                   jax.ShapeDtypeStruct((2, N, D), jnp.float32)],
    )(feats, wqt, wkt, wvt, b3)


GB = 200
EPT = E // NS
GCH = EPT // GB


def _gather_body(q2, kv2, cidx, ridx, pv2, cbuf, rbuf, qbuf, kvbuf,
                 semq, semk, semw):
    g = lax.axis_index("c")
    s = lax.axis_index("s")

    def chunk(t, _):
        base = s * EPT + t * GB
        pltpu.sync_copy(cidx.at[pl.ds(base, GB)], cbuf)
        pltpu.sync_copy(ridx.at[pl.ds(base, GB)], rbuf)
        dq = pltpu.async_copy(q2.at[g].at[cbuf], qbuf, semq)
        @pl.when(t > 0)
        def _():
            pltpu.make_async_copy(pv2.at[g, pl.ds(0, GB)], kvbuf, semw).wait()

        dkv = pltpu.async_copy(kv2.at[g].at[rbuf], kvbuf, semk)
        dq.wait()
        dkv.wait()

        def row(i, _):
            for u in range(4):
                for j in range(HALF // 16):
                    sl = pl.ds(j * 16, 16)
                    kvbuf[i + u, sl] = kvbuf[i + u, sl] * qbuf[i + u, sl]
            return 0

        lax.fori_loop(0, GB // 4, lambda i, a: row(i * 4, a), 0)
        pltpu.async_copy(kvbuf, pv2.at[g, pl.ds(base, GB)], semw)
        return 0

    lax.fori_loop(0, GCH, chunk, 0)
    pltpu.make_async_copy(pv2.at[g, pl.ds(0, GB)], kvbuf, semw).wait()


def _gather_qkv(q2, kv2, cidx, ridx):
    fn = pl.kernel(
        _gather_body,
        out_type=[jax.ShapeDtypeStruct((2, E, D), jnp.float32)],
        mesh=_MESH(),
        scratch_types=[
            pltpu.VMEM((GB,), jnp.int32),
            pltpu.VMEM((GB,), jnp.int32),
            pltpu.VMEM((GB, HALF), jnp.float32),
            pltpu.VMEM((GB, D), jnp.float32),
            pltpu.SemaphoreType.DMA,
            pltpu.SemaphoreType.DMA,
            pltpu.SemaphoreType.DMA,
        ],
    )
    (pv2,) = fn(q2, kv2, cidx, ridx)
    return pv2


BE = 2000


def _dot2(x, m_ref):
    xh = x.astype(jnp.bfloat16)
    xl = (x - xh.astype(jnp.float32)).astype(jnp.bfloat16)
    m = m_ref[...]
    return (jnp.dot(xh, m, preferred_element_type=jnp.float32)
            + jnp.dot(xl, m, preferred_element_type=jnp.float32))


def _score_body(pv_ref, b0_ref, b1_ref, pexp_ref, o_ref, w_ref):
    pv0 = pv_ref[0]
    pv1 = pv_ref[1]
    s = (_dot2(pv0[:, :HALF], b0_ref) + _dot2(pv1[:, :HALF], b1_ref)) * (
        1.0 / math.sqrt(HDIM))
    ex = jnp.exp(s)
    aexp = _dot2(ex, pexp_ref)
    w0 = aexp[:, :HALF]
    w1 = aexp[:, HALF:]
    o_ref[0] = w0 * pv0[:, HALF:]
    o_ref[1] = w1 * pv1[:, HALF:]
    w_ref[0] = w0
    w_ref[1] = w1


def _score_weight(pv2, b0, b1, pexp):
    grid = (E // BE,)
    oshape = jax.ShapeDtypeStruct((2, E, HALF), jnp.float32)
    ospec = pl.BlockSpec((2, BE, HALF), lambda i: (0, i, 0))
    return pl.pallas_call(
        _score_body, grid=grid,
        in_specs=[pl.BlockSpec((2, BE, D), lambda i: (0, i, 0)),
                  pl.BlockSpec((HALF, 16), lambda i: (0, 0)),
                  pl.BlockSpec((HALF, 16), lambda i: (0, 0)),
                  pl.BlockSpec((16, D), lambda i: (0, 0))],
        out_specs=[ospec, ospec],
        out_shape=[oshape, oshape],
    )(pv2, b0, b1, pexp)


AB = 200
ACH = E // NS // AB


def _agg_body(o, w, cidx, z128, agg2, denw, acc, cbuf, obuf, semr):
    g = lax.axis_index("c")
    s = lax.axis_index("s")
    rows = NP // NS
    rsl = pl.ds(s * rows, rows)

    for src_ref, dst_ref in ((o, agg2), (w, denw)):
        pltpu.sync_copy(z128.at[rsl], acc.at[rsl])
        plsc.subcore_barrier()

        def scat(t, _, src_ref=src_ref):
            base = s * (E // NS) + t * AB
            dr = pltpu.async_copy(src_ref.at[g].at[pl.ds(base, AB)], obuf,
                                  semr)
            pltpu.sync_copy(cidx.at[pl.ds(base, AB)], cbuf)
            dr.wait()
            pltpu.sync_copy(obuf, acc.at[cbuf], add=True)
            return 0

        lax.fori_loop(0, ACH, scat, 0)
        plsc.subcore_barrier()
        pltpu.sync_copy(acc.at[rsl], dst_ref.at[g, rsl])
        plsc.subcore_barrier()


def _agg(o, w, cidx, z128):
    shp = jax.ShapeDtypeStruct((NC, NP, HALF), jnp.float32)
    fn = pl.kernel(
        _agg_body,
        out_type=[shp, shp],
        mesh=_MESH(),
        scratch_types=[
            pltpu.VMEM_SHARED((NP, HALF), jnp.float32),
            pltpu.VMEM((AB,), jnp.int32),
            pltpu.VMEM((AB, HALF), jnp.float32),
            pltpu.SemaphoreType.DMA,
        ],
    )
    return fn(o, w, cidx, z128)


def _final_body(a_ref, d_ref, w0_ref, w1_ref, b_ref, out_ref):
    n0 = a_ref[0] / (d_ref[0] + 1e-16)
    n1 = a_ref[1] / (d_ref[1] + 1e-16)
    acc = jnp.dot(n0, w0_ref[...], preferred_element_type=jnp.float32, precision=lax.Precision.HIGHEST)
    acc += jnp.dot(n1, w1_ref[...], preferred_element_type=jnp.float32, precision=lax.Precision.HIGHEST)
    out_ref[...] = acc + b_ref[...]


def _final_proj(agg2, denw, wot0, wot1, bo2):
    grid = (N // BN,)
    aspec = pl.BlockSpec((2, BN, HALF), lambda i: (0, i, 0))
    return pl.pallas_call(
        _final_body, grid=grid,
        in_specs=[aspec, aspec,
                  pl.BlockSpec((HALF, D), lambda i: (0, 0)),
                  pl.BlockSpec((HALF, D), lambda i: (0, 0)),
                  pl.BlockSpec((1, D), lambda i: (0, 0))],
        out_specs=pl.BlockSpec((BN, D), lambda i: (i, 0)),
        out_shape=jax.ShapeDtypeStruct((N, D), jnp.float32),
    )(agg2, denw, wot0, wot1, bo2)


def kernel(feats, edge_index, edge_attr, Wq, bq, Wk, bk, Wv, bv, Wo, bo):
    del edge_attr
    r = edge_index[:, 0]
    c = edge_index[:, 1]

    wqt = Wq.T
    wkt = Wk.T
    wvt = Wv.T
    b3 = jnp.stack([bq, bk, bv], axis=0)

    bd = np.zeros((D, 16), np.float32)
    for h in range(NHEAD):
        bd[h * HDIM:(h + 1) * HDIM, h] = 1.0
    b0 = jnp.asarray(bd[:HALF], dtype=jnp.bfloat16)
    b1 = jnp.asarray(bd[HALF:], dtype=jnp.bfloat16)

    pexp = np.zeros((16, D), np.float32)
    for h in range(NHEAD):
        pexp[h, h * HDIM:(h + 1) * HDIM] = 1.0
    pexp = jnp.asarray(pexp, dtype=jnp.bfloat16)

    z128 = jnp.zeros((NP, HALF), jnp.float32)

    q2, kv2 = _qkv_proj(feats, wqt, wkt, wvt, b3)
    pv2 = _gather_qkv(q2, kv2, c, r)
    o, w = _score_weight(pv2, b0, b1, pexp)
    agg2, denw = _agg(o, w, c, z128)

    wot = Wo.T
    return _final_proj(agg2, denw, wot[:HALF], wot[HALF:], bo.reshape(1, D))


# speedup vs baseline: 1.2335x; 1.2335x over previous
import functools
import math

import jax
import jax.numpy as jnp
import numpy as np
from jax import lax
from jax.experimental import pallas as pl
from jax.experimental.pallas import tpu as pltpu
from jax.experimental.pallas import tpu_sc as plsc

N = 10000
E = 160000
D = 256
NHEAD = 8
HDIM = D // NHEAD

NC = 2
NS = 16
HALF = D // 2
NP = 10112

_MESH = functools.partial(
    plsc.VectorSubcoreMesh, core_axis_name="c", subcore_axis_name="s",
    num_cores=NC, num_subcores=NS)


BN = 1000


def _qkv_body(x_ref, wq_ref, wk_ref, wv_ref, b_ref, q_ref, k_ref, v_ref):
    x = x_ref[...]
    b = b_ref[...]
    xq = jnp.dot(x, wq_ref[...], preferred_element_type=jnp.float32, precision=lax.Precision.HIGHEST) + b[0:1, :]
    xk = jnp.dot(x, wk_ref[...], preferred_element_type=jnp.float32, precision=lax.Precision.HIGHEST) + b[1:2, :]
    xv = jnp.dot(x, wv_ref[...], preferred_element_type=jnp.float32, precision=lax.Precision.HIGHEST) + b[2:3, :]
    q_ref[0] = xq[:, :HALF]
    q_ref[1] = xq[:, HALF:]
    k_ref[0] = xk[:, :HALF]
    k_ref[1] = xk[:, HALF:]
    v_ref[0] = xv[:, :HALF]
    v_ref[1] = xv[:, HALF:]


def _qkv_proj(feats, wqt, wkt, wvt, b3):
    grid = (N // BN,)
    spec_x = pl.BlockSpec((BN, D), lambda i: (i, 0))
    spec_w = pl.BlockSpec((D, D), lambda i: (0, 0))
    spec_b = pl.BlockSpec((3, D), lambda i: (0, 0))
    out = pl.BlockSpec((2, BN, HALF), lambda i: (0, i, 0))
    shp = jax.ShapeDtypeStruct((2, N, HALF), jnp.float32)
    return pl.pallas_call(
        _qkv_body, grid=grid,
        in_specs=[spec_x, spec_w, spec_w, spec_w, spec_b],
        out_specs=[out, out, out],
        out_shape=[shp, shp, shp],
    )(feats, wqt, wkt, wvt, b3)


GB = 200
EPT = E // NS
GCH = EPT // GB


def _gather_body(q2, k2, v2, cidx, ridx, prod2, vr2, cbuf, rbuf, qbuf, kbuf,
                 vbuf, semq, semk, semv, semw):
    g = lax.axis_index("c")
    s = lax.axis_index("s")

    def drain2():
        pltpu.make_async_copy(prod2.at[g, pl.ds(0, GB)], qbuf, semw).wait()
        pltpu.make_async_copy(prod2.at[g, pl.ds(0, GB)], qbuf, semw).wait()

    def chunk(t, _):
        base = s * EPT + t * GB
        pltpu.sync_copy(cidx.at[pl.ds(base, GB)], cbuf)
        pltpu.sync_copy(ridx.at[pl.ds(base, GB)], rbuf)

        @pl.when(t > 0)
        def _():
            drain2()

        dq = pltpu.async_copy(q2.at[g].at[cbuf], qbuf, semq)
        dk = pltpu.async_copy(k2.at[g].at[rbuf], kbuf, semk)
        dv = pltpu.async_copy(v2.at[g].at[rbuf], vbuf, semv)
        dq.wait()
        dk.wait()

        def row(i, _):
            for u in range(4):
                for j in range(HALF // 16):
                    sl = pl.ds(j * 16, 16)
                    qbuf[i + u, sl] = qbuf[i + u, sl] * kbuf[i + u, sl]
            return 0

        lax.fori_loop(0, GB // 4, lambda i, a: row(i * 4, a), 0)
        pltpu.async_copy(qbuf, prod2.at[g, pl.ds(base, GB)], semw)
        dv.wait()
        pltpu.async_copy(vbuf, vr2.at[g, pl.ds(base, GB)], semw)
        return 0

    lax.fori_loop(0, GCH, chunk, 0)
    drain2()


def _gather_qkv(q2, k2, v2, cidx, ridx):
    shp = jax.ShapeDtypeStruct((2, E, HALF), jnp.float32)
    fn = pl.kernel(
        _gather_body,
        out_type=[shp, shp],
        mesh=_MESH(),
        scratch_types=[
            pltpu.VMEM((GB,), jnp.int32),
            pltpu.VMEM((GB,), jnp.int32),
            pltpu.VMEM((GB, HALF), jnp.float32),
            pltpu.VMEM((GB, HALF), jnp.float32),
            pltpu.VMEM((GB, HALF), jnp.float32),
            pltpu.SemaphoreType.DMA,
            pltpu.SemaphoreType.DMA,
            pltpu.SemaphoreType.DMA,
            pltpu.SemaphoreType.DMA,
        ],
    )
    return fn(q2, k2, v2, cidx, ridx)


BE = 2000


def _dot2(x, m_ref):
    xh = x.astype(jnp.bfloat16)
    xl = (x - xh.astype(jnp.float32)).astype(jnp.bfloat16)
    m = m_ref[...]
    return (jnp.dot(xh, m, preferred_element_type=jnp.float32)
            + jnp.dot(xl, m, preferred_element_type=jnp.float32))


def _score_body(p2_ref, vr2_ref, b0_ref, b1_ref, pexp_ref, o_ref, w_ref):
    s = (_dot2(p2_ref[0], b0_ref) + _dot2(p2_ref[1], b1_ref)) * (
        1.0 / math.sqrt(HDIM))
    ex = jnp.exp(s)
    aexp = _dot2(ex, pexp_ref)
    w0 = aexp[:, :HALF]
    w1 = aexp[:, HALF:]
    o_ref[0] = w0 * vr2_ref[0]
    o_ref[1] = w1 * vr2_ref[1]
    w_ref[0] = w0
    w_ref[1] = w1


def _score_weight(prod2, vr2, b0, b1, pexp):
    grid = (E // BE,)
    oshape = jax.ShapeDtypeStruct((2, E, HALF), jnp.float32)
    ospec = pl.BlockSpec((2, BE, HALF), lambda i: (0, i, 0))
    return pl.pallas_call(
        _score_body, grid=grid,
        in_specs=[ospec, ospec,
                  pl.BlockSpec((HALF, 16), lambda i: (0, 0)),
                  pl.BlockSpec((HALF, 16), lambda i: (0, 0)),
                  pl.BlockSpec((16, D), lambda i: (0, 0))],
        out_specs=[ospec, ospec],
        out_shape=[oshape, oshape],
    )(prod2, vr2, b0, b1, pexp)


AB = 200
ACH = E // NS // AB


def _agg_body(o, w, cidx, z128, agg2, denw, acc, cbuf, obuf, semr):
    g = lax.axis_index("c")
    s = lax.axis_index("s")
    rows = NP // NS
    rsl = pl.ds(s * rows, rows)

    for src_ref, dst_ref in ((o, agg2), (w, denw)):
        pltpu.sync_copy(z128.at[rsl], acc.at[rsl])
        plsc.subcore_barrier()

        def scat(t, _, src_ref=src_ref):
            base = s * (E // NS) + t * AB
            dr = pltpu.async_copy(src_ref.at[g].at[pl.ds(base, AB)], obuf,
                                  semr)
            pltpu.sync_copy(cidx.at[pl.ds(base, AB)], cbuf)
            dr.wait()
            pltpu.sync_copy(obuf, acc.at[cbuf], add=True)
            return 0

        lax.fori_loop(0, ACH, scat, 0)
        plsc.subcore_barrier()
        pltpu.sync_copy(acc.at[rsl], dst_ref.at[g, rsl])
        plsc.subcore_barrier()


def _agg(o, w, cidx, z128):
    shp = jax.ShapeDtypeStruct((NC, NP, HALF), jnp.float32)
    fn = pl.kernel(
        _agg_body,
        out_type=[shp, shp],
        mesh=_MESH(),
        scratch_types=[
            pltpu.VMEM_SHARED((NP, HALF), jnp.float32),
            pltpu.VMEM((AB,), jnp.int32),
            pltpu.VMEM((AB, HALF), jnp.float32),
            pltpu.SemaphoreType.DMA,
        ],
    )
    return fn(o, w, cidx, z128)


def _final_body(a_ref, d_ref, w0_ref, w1_ref, b_ref, out_ref):
    n0 = a_ref[0] / (d_ref[0] + 1e-16)
    n1 = a_ref[1] / (d_ref[1] + 1e-16)
    acc = jnp.dot(n0, w0_ref[...], preferred_element_type=jnp.float32, precision=lax.Precision.HIGHEST)
    acc += jnp.dot(n1, w1_ref[...], preferred_element_type=jnp.float32, precision=lax.Precision.HIGHEST)
    out_ref[...] = acc + b_ref[...]


def _final_proj(agg2, denw, wot0, wot1, bo2):
    grid = (N // BN,)
    aspec = pl.BlockSpec((2, BN, HALF), lambda i: (0, i, 0))
    return pl.pallas_call(
        _final_body, grid=grid,
        in_specs=[aspec, aspec,
                  pl.BlockSpec((HALF, D), lambda i: (0, 0)),
                  pl.BlockSpec((HALF, D), lambda i: (0, 0)),
                  pl.BlockSpec((1, D), lambda i: (0, 0))],
        out_specs=pl.BlockSpec((BN, D), lambda i: (i, 0)),
        out_shape=jax.ShapeDtypeStruct((N, D), jnp.float32),
    )(agg2, denw, wot0, wot1, bo2)


def kernel(feats, edge_index, edge_attr, Wq, bq, Wk, bk, Wv, bv, Wo, bo):
    del edge_attr
    r = edge_index[:, 0]
    c = edge_index[:, 1]

    wqt = Wq.T
    wkt = Wk.T
    wvt = Wv.T
    b3 = jnp.stack([bq, bk, bv], axis=0)

    bd = np.zeros((D, 16), np.float32)
    for h in range(NHEAD):
        bd[h * HDIM:(h + 1) * HDIM, h] = 1.0
    b0 = jnp.asarray(bd[:HALF], dtype=jnp.bfloat16)
    b1 = jnp.asarray(bd[HALF:], dtype=jnp.bfloat16)

    pexp = np.zeros((16, D), np.float32)
    for h in range(NHEAD):
        pexp[h, h * HDIM:(h + 1) * HDIM] = 1.0
    pexp = jnp.asarray(pexp, dtype=jnp.bfloat16)

    z128 = jnp.zeros((NP, HALF), jnp.float32)

    q2, k2, v2 = _qkv_proj(feats, wqt, wkt, wvt, b3)
    prod2, vr2 = _gather_qkv(q2, k2, v2, c, r)
    o, w = _score_weight(prod2, vr2, b0, b1, pexp)
    agg2, denw = _agg(o, w, c, z128)

    wot = Wo.T
    return _final_proj(agg2, denw, wot[:HALF], wot[HALF:], bo.reshape(1, D))
